# baseline (speedup 1.0000x reference)
import functools

import jax
import jax.numpy as jnp
from jax import lax
from jax.experimental import pallas as pl
from jax.experimental.pallas import tpu as pltpu
from jax.experimental.pallas import tpu_sc as plsc

_L = 16
_NUM_WORKERS = 32


_CHUNK = 4096


def _encode_body(u_hbm, y_hbm, out_hbm, out2_hbm, u_v, tab_v,
                 y_v0, y_v1, o_v0, o_v1, in_s0, in_s1, out_s0, out_s1,
                 out2_s0, out2_s1):
    t = u_v.shape[0]
    per_w = _CHUNK * 8
    nchunks = per_w // _CHUNK
    y_bufs, o_bufs = (y_v0, y_v1), (o_v0, o_v1)
    in_sems, out_sems = (in_s0, in_s1), (out_s0, out_s1)
    out2_sems = (out2_s0, out2_s1)
    wid = lax.axis_index("s") * 2 + lax.axis_index("c")
    base = wid * per_w

    in_copies = [None] * nchunks
    out_copies = [None] * nchunks
    out2_copies = [None] * nchunks
    in_copies[0] = pltpu.make_async_copy(
        y_hbm.at[pl.ds(base, _CHUNK)], y_bufs[0], in_sems[0])
    in_copies[0].start()
    pltpu.sync_copy(u_hbm, u_v)

    zeros = jnp.zeros((_L,), jnp.int32)
    for c in range(t // _L):
        tab_v[pl.ds(c * _L, _L)] = zeros
    ones = jnp.ones((_L,), jnp.int32)
    for c in range(t // _L):
        plsc.addupdate_scatter(tab_v, [u_v[pl.ds(c * _L, _L)]], ones)

    carry = jnp.int32(0)
    for c in range(t // _L):
        h = tab_v[pl.ds(c * _L, _L)]
        incl = plsc.cumsum(h)
        tab_v[pl.ds(c * _L, _L)] = incl - h + carry
        carry = carry + jnp.sum(h)

    for g in range(nchunks):
        b = g & 1
        in_copies[g].wait()
        if g + 1 < nchunks:
            in_copies[g + 1] = pltpu.make_async_copy(
                y_hbm.at[pl.ds(base + (g + 1) * _CHUNK, _CHUNK)],
                y_bufs[(g + 1) & 1], in_sems[(g + 1) & 1])
            in_copies[g + 1].start()
        if g >= 2:
            out_copies[g - 2].wait()
            out2_copies[g - 2].wait()

        y_v, o_v = y_bufs[b], o_bufs[b]

        @plsc.parallel_loop(0, _CHUNK, _L, unroll=16)
        def _gbody(i):
            sl = pl.ds(i, _L)
            o_v[sl] = plsc.load_gather(tab_v, [y_v[sl]])

        out_copies[g] = pltpu.make_async_copy(
            o_v, out_hbm.at[pl.ds(base + g * _CHUNK, _CHUNK)], out_sems[b])
        out_copies[g].start()
        out2_copies[g] = pltpu.make_async_copy(
            o_v, out2_hbm.at[pl.ds(base + g * _CHUNK, _CHUNK)], out2_sems[b])
        out2_copies[g].start()
    for g in (nchunks - 2, nchunks - 1):
        out_copies[g].wait()
        out2_copies[g].wait()


def kernel(y_n, unique_cell_types):
    y = y_n.astype(jnp.int32)
    u = unique_cell_types.astype(jnp.int32)
    n = y.shape[0]
    t = u.shape[0]
    per_w = n // _NUM_WORKERS
    assert per_w == _CHUNK * 8
    mesh = plsc.VectorSubcoreMesh(core_axis_name="c", subcore_axis_name="s")
    run = functools.partial(
        pl.kernel,
        mesh=mesh,
        compiler_params=pltpu.CompilerParams(needs_layout_passes=False),
        out_type=(jax.ShapeDtypeStruct((n,), jnp.int32),
                  jax.ShapeDtypeStruct((n,), jnp.int32)),
        scratch_types=[
            pltpu.VMEM((t,), jnp.int32),
            pltpu.VMEM((t,), jnp.int32),
            pltpu.VMEM((_CHUNK,), jnp.int32),
            pltpu.VMEM((_CHUNK,), jnp.int32),
            pltpu.VMEM((_CHUNK,), jnp.int32),
            pltpu.VMEM((_CHUNK,), jnp.int32),
            pltpu.SemaphoreType.DMA,
            pltpu.SemaphoreType.DMA,
            pltpu.SemaphoreType.DMA,
            pltpu.SemaphoreType.DMA,
            pltpu.SemaphoreType.DMA,
            pltpu.SemaphoreType.DMA,
        ],
    )(_encode_body)
    out, out2 = run(u, y)
    return (out, out2)


# speedup vs baseline: 1.1102x; 1.1102x over previous
import functools

import jax
import jax.numpy as jnp
from jax import lax
from jax.experimental import pallas as pl
from jax.experimental.pallas import tpu as pltpu
from jax.experimental.pallas import tpu_sc as plsc

_L = 16
_NUM_WORKERS = 32


_CHUNK = 8192


def _encode_body(u_hbm, y_hbm, out_hbm, out2_hbm, u_v, tab_v,
                 y_v0, y_v1, o_v0, o_v1, in_s0, in_s1, out_s0, out_s1,
                 out2_s0, out2_s1):
    t = u_v.shape[0]
    per_w = _CHUNK * 4
    nchunks = per_w // _CHUNK
    y_bufs, o_bufs = (y_v0, y_v1), (o_v0, o_v1)
    in_sems, out_sems = (in_s0, in_s1), (out_s0, out_s1)
    out2_sems = (out2_s0, out2_s1)
    wid = lax.axis_index("s") * 2 + lax.axis_index("c")
    base = wid * per_w

    in_copies = [None] * nchunks
    out_copies = [None] * nchunks
    out2_copies = [None] * nchunks
    in_copies[0] = pltpu.make_async_copy(
        y_hbm.at[pl.ds(base, _CHUNK)], y_bufs[0], in_sems[0])
    in_copies[0].start()
    pltpu.sync_copy(u_hbm, u_v)

    zeros = jnp.zeros((_L,), jnp.int32)
    for c in range(t // _L):
        tab_v[pl.ds(c * _L, _L)] = zeros
    ones = jnp.ones((_L,), jnp.int32)
    for c in range(t // _L):
        plsc.addupdate_scatter(tab_v, [u_v[pl.ds(c * _L, _L)]], ones)

    carry = jnp.int32(0)
    for c in range(t // _L):
        h = tab_v[pl.ds(c * _L, _L)]
        incl = plsc.cumsum(h)
        tab_v[pl.ds(c * _L, _L)] = incl - h + carry
        carry = carry + jnp.sum(h)

    for g in range(nchunks):
        b = g & 1
        in_copies[g].wait()
        if g + 1 < nchunks:
            in_copies[g + 1] = pltpu.make_async_copy(
                y_hbm.at[pl.ds(base + (g + 1) * _CHUNK, _CHUNK)],
                y_bufs[(g + 1) & 1], in_sems[(g + 1) & 1])
            in_copies[g + 1].start()
        if g >= 2:
            out_copies[g - 2].wait()
            out2_copies[g - 2].wait()

        y_v, o_v = y_bufs[b], o_bufs[b]

        @plsc.parallel_loop(0, _CHUNK, _L, unroll=16)
        def _gbody(i):
            sl = pl.ds(i, _L)
            o_v[sl] = plsc.load_gather(tab_v, [y_v[sl]])

        out_copies[g] = pltpu.make_async_copy(
            o_v, out_hbm.at[pl.ds(base + g * _CHUNK, _CHUNK)], out_sems[b])
        out_copies[g].start()
        out2_copies[g] = pltpu.make_async_copy(
            o_v, out2_hbm.at[pl.ds(base + g * _CHUNK, _CHUNK)], out2_sems[b])
        out2_copies[g].start()
    for g in (nchunks - 2, nchunks - 1):
        out_copies[g].wait()
        out2_copies[g].wait()


def kernel(y_n, unique_cell_types):
    y = y_n.astype(jnp.int32)
    u = unique_cell_types.astype(jnp.int32)
    n = y.shape[0]
    t = u.shape[0]
    per_w = n // _NUM_WORKERS
    assert per_w == _CHUNK * 4
    mesh = plsc.VectorSubcoreMesh(core_axis_name="c", subcore_axis_name="s")
    run = functools.partial(
        pl.kernel,
        mesh=mesh,
        compiler_params=pltpu.CompilerParams(needs_layout_passes=False),
        out_type=(jax.ShapeDtypeStruct((n,), jnp.int32),
                  jax.ShapeDtypeStruct((n,), jnp.int32)),
        scratch_types=[
            pltpu.VMEM((t,), jnp.int32),
            pltpu.VMEM((t,), jnp.int32),
            pltpu.VMEM((_CHUNK,), jnp.int32),
            pltpu.VMEM((_CHUNK,), jnp.int32),
            pltpu.VMEM((_CHUNK,), jnp.int32),
            pltpu.VMEM((_CHUNK,), jnp.int32),
            pltpu.SemaphoreType.DMA,
            pltpu.SemaphoreType.DMA,
            pltpu.SemaphoreType.DMA,
            pltpu.SemaphoreType.DMA,
            pltpu.SemaphoreType.DMA,
            pltpu.SemaphoreType.DMA,
        ],
    )(_encode_body)
    out, out2 = run(u, y)
    return (out, out2)


# speedup vs baseline: 1.1843x; 1.0668x over previous
import functools

import jax
import jax.numpy as jnp
from jax import lax
from jax.experimental import pallas as pl
from jax.experimental.pallas import tpu as pltpu
from jax.experimental.pallas import tpu_sc as plsc

_L = 16
_NUM_WORKERS = 32


_CHUNK = 8192


def _encode_body(u_hbm, y_hbm, out_hbm, out2_hbm, u_v, tab_v,
                 b0, b1, b2, b3, i_s0, i_s1, i_s2, i_s3,
                 o_s0, o_s1, o_s2, o_s3, q_s0, q_s1, q_s2, q_s3):
    t = u_v.shape[0]
    nchunks = 4
    per_w = _CHUNK * nchunks
    bufs = (b0, b1, b2, b3)
    in_sems = (i_s0, i_s1, i_s2, i_s3)
    out_sems = (o_s0, o_s1, o_s2, o_s3)
    out2_sems = (q_s0, q_s1, q_s2, q_s3)
    wid = lax.axis_index("s") * 2 + lax.axis_index("c")
    base = wid * per_w

    in_copies = []
    for g in range(nchunks):
        cp = pltpu.make_async_copy(
            y_hbm.at[pl.ds(base + g * _CHUNK, _CHUNK)], bufs[g], in_sems[g])
        cp.start()
        in_copies.append(cp)
    pltpu.sync_copy(u_hbm, u_v)

    zeros = jnp.zeros((_L,), jnp.int32)
    for c in range(t // _L):
        tab_v[pl.ds(c * _L, _L)] = zeros
    ones = jnp.ones((_L,), jnp.int32)
    for c in range(t // _L):
        plsc.addupdate_scatter(tab_v, [u_v[pl.ds(c * _L, _L)]], ones)

    carry = jnp.int32(0)
    for c in range(t // _L):
        h = tab_v[pl.ds(c * _L, _L)]
        incl = plsc.cumsum(h)
        tab_v[pl.ds(c * _L, _L)] = incl - h + carry
        carry = carry + jnp.sum(h)

    out_copies = []
    for g in range(nchunks):
        in_copies[g].wait()
        buf = bufs[g]

        @plsc.parallel_loop(0, _CHUNK, _L, unroll=8)
        def _gbody(i):
            sl = pl.ds(i, _L)
            buf[sl] = plsc.load_gather(tab_v, [buf[sl]])

        for dst, sem in ((out_hbm, out_sems[g]), (out2_hbm, out2_sems[g])):
            cp = pltpu.make_async_copy(
                buf, dst.at[pl.ds(base + g * _CHUNK, _CHUNK)], sem)
            cp.start()
            out_copies.append(cp)
    for cp in out_copies:
        cp.wait()


def kernel(y_n, unique_cell_types):
    y = y_n.astype(jnp.int32)
    u = unique_cell_types.astype(jnp.int32)
    n = y.shape[0]
    t = u.shape[0]
    per_w = n // _NUM_WORKERS
    assert per_w == _CHUNK * 4
    mesh = plsc.VectorSubcoreMesh(core_axis_name="c", subcore_axis_name="s")
    run = functools.partial(
        pl.kernel,
        mesh=mesh,
        compiler_params=pltpu.CompilerParams(needs_layout_passes=False),
        out_type=(jax.ShapeDtypeStruct((n,), jnp.int32),
                  jax.ShapeDtypeStruct((n,), jnp.int32)),
        scratch_types=[
            pltpu.VMEM((t,), jnp.int32),
            pltpu.VMEM((t,), jnp.int32),
            pltpu.VMEM((_CHUNK,), jnp.int32),
            pltpu.VMEM((_CHUNK,), jnp.int32),
            pltpu.VMEM((_CHUNK,), jnp.int32),
            pltpu.VMEM((_CHUNK,), jnp.int32),
            pltpu.SemaphoreType.DMA,
            pltpu.SemaphoreType.DMA,
            pltpu.SemaphoreType.DMA,
            pltpu.SemaphoreType.DMA,
            pltpu.SemaphoreType.DMA,
            pltpu.SemaphoreType.DMA,
            pltpu.SemaphoreType.DMA,
            pltpu.SemaphoreType.DMA,
            pltpu.SemaphoreType.DMA,
            pltpu.SemaphoreType.DMA,
            pltpu.SemaphoreType.DMA,
            pltpu.SemaphoreType.DMA,
        ],
    )(_encode_body)
    out, out2 = run(u, y)
    return (out, out2)
